# baseline (device time: 46790 ns/iter reference)
import os

import jax
import jax.numpy as jnp
from jax import lax
from jax.experimental import pallas as pl
from jax.experimental.pallas import tpu as pltpu

N_DEV = 4
S = 2
_SKIP_AG = bool(os.environ.get("SKIP_AG"))


def kernel(t, W):
    m_per, k = t.shape
    _, n = W.shape
    chunk = m_per // N_DEV
    sh = chunk // S
    kh = k // 2
    nh = n // 2
    bf16 = jnp.bfloat16
    H = N_DEV - 1

    def body(t_ref, w_ref, out_ref,
             stage, recv, pre_own, pre_comb,
             rs_ssem, rs_rsem,
             ag_r_ssem, ag_r_rsem, ag_l_ssem, ag_l_rsem):
        my = lax.axis_index("i")
        left = lax.rem(my + N_DEV - 1, N_DEV)
        right = lax.rem(my + 1, N_DEV)

        def t_sub(idx, s, lo):
            rows = pl.ds(idx * chunk + s * sh, sh)
            if lo:
                return t_ref[rows, :kh].astype(bf16)
            return t_ref[rows, kh:].astype(bf16)

        def copy(src, dst, ssem, rsem, dev):
            return pltpu.make_async_remote_copy(
                src_ref=src, dst_ref=dst, send_sem=ssem, recv_sem=rsem,
                device_id=(dev,), device_id_type=pl.DeviceIdType.MESH,
            )

        def rs_start(s, j):
            dev = {0: left, 1: right, 2: right, 3: left, 4: left, 5: right}[j]
            c = copy(stage.at[s, j], recv.at[s, j],
                     rs_ssem.at[s, j], rs_rsem.at[s, j], dev)
            c.start()
            return c

        def out_rows(idx, s):
            return pl.ds(idx * chunk + s * sh, sh)

        def start_ag_r(s, h):
            idx = lax.rem(my - h + 2 * N_DEV, N_DEV)
            r = copy(out_ref.at[out_rows(idx, s), pl.ds(0, nh)],
                     out_ref.at[out_rows(idx, s), pl.ds(0, nh)],
                     ag_r_ssem.at[s, h], ag_r_rsem.at[s, h], right)
            r.start()
            return r

        def start_ag_l(s, h):
            idx = lax.rem(my + h, N_DEV)
            l = copy(out_ref.at[out_rows(idx, s), pl.ds(nh, nh)],
                     out_ref.at[out_rows(idx, s), pl.ds(nh, nh)],
                     ag_l_ssem.at[s, h], ag_l_rsem.at[s, h], left)
            l.start()
            return l

        c_m2 = lax.rem(my - 2 + N_DEV, N_DEV)
        c_m1 = lax.rem(my - 1 + N_DEV, N_DEV)
        c_p1 = lax.rem(my + 1, N_DEV)
        for s in range(S):
            stage[s, 0, :, :] = t_sub(c_m2, s, True)
            stage[s, 1, :, :] = t_sub(c_m2, s, False)
            stage[s, 2, :, :] = t_sub(c_p1, s, True)
            stage[s, 3, :, :] = t_sub(c_m1, s, False)

        barrier_sem = pltpu.get_barrier_semaphore()
        for nbr in (left, right):
            pl.semaphore_signal(
                barrier_sem, inc=1,
                device_id=(nbr,), device_id_type=pl.DeviceIdType.MESH,
            )
        pl.semaphore_wait(barrier_sem, 2)

        rs = {}
        for s in range(S):
            rs[s, 0] = rs_start(s, 0)
            rs[s, 1] = rs_start(s, 1)
        for s in range(S):
            rs[s, 2] = rs_start(s, 2)
            rs[s, 3] = rs_start(s, 3)

        for s in range(S):
            pre_comb[s, 0, :, :] = t_sub(c_m1, s, True)
            pre_comb[s, 1, :, :] = t_sub(c_p1, s, False)
            pre_own[s, 0, :, :] = t_sub(my, s, True)
            pre_own[s, 1, :, :] = t_sub(my, s, False)
        w_bf = w_ref[:, :].astype(bf16)

        for s in range(S):
            rs[s, 0].wait()
            stage[s, 4, :, :] = pre_comb[s, 0] + recv[s, 0]
            rs[s, 4] = rs_start(s, 4)
            rs[s, 1].wait()
            stage[s, 5, :, :] = pre_comb[s, 1] + recv[s, 1]
            rs[s, 5] = rs_start(s, 5)

        ag = {}
        for s in range(S):
            rs[s, 2].wait()
            rs[s, 4].wait()
            sum_lo = pre_own[s, 0] + recv[s, 2] + recv[s, 4]
            rs[s, 3].wait()
            rs[s, 5].wait()
            sum_hi = pre_own[s, 1] + recv[s, 3] + recv[s, 5]
            out_sub = (
                jnp.dot(sum_lo, w_bf[:kh, :],
                        preferred_element_type=jnp.float32)
                + jnp.dot(sum_hi, w_bf[kh:, :],
                          preferred_element_type=jnp.float32)
            ).astype(bf16)
            out_ref[out_rows(my, s), :] = out_sub
            if not _SKIP_AG:
                ag[s, 0] = (start_ag_r(s, 0), start_ag_l(s, 0))

        if not _SKIP_AG:
            for h in range(H - 1):
                for s in range(S):
                    r, l = ag[s, h]
                    r.wait()
                    nr = start_ag_r(s, h + 1)
                    l.wait()
                    nl = start_ag_l(s, h + 1)
                    ag[s, h + 1] = (nr, nl)
            for s in range(S):
                r, l = ag[s, H - 1]
                r.wait()
                l.wait()

    return pl.pallas_call(
        body,
        out_shape=jax.ShapeDtypeStruct((m_per, n), bf16),
        in_specs=[
            pl.BlockSpec(memory_space=pltpu.VMEM),
            pl.BlockSpec(memory_space=pltpu.VMEM),
        ],
        out_specs=pl.BlockSpec(memory_space=pltpu.VMEM),
        scratch_shapes=[
            pltpu.VMEM((S, 6, sh, kh), bf16),
            pltpu.VMEM((S, 6, sh, kh), bf16),
            pltpu.VMEM((S, 2, sh, kh), bf16),
            pltpu.VMEM((S, 2, sh, kh), bf16),
            pltpu.SemaphoreType.DMA((S, 6)),
            pltpu.SemaphoreType.DMA((S, 6)),
            pltpu.SemaphoreType.DMA((S, H)),
            pltpu.SemaphoreType.DMA((S, H)),
            pltpu.SemaphoreType.DMA((S, H)),
            pltpu.SemaphoreType.DMA((S, H)),
        ],
        compiler_params=pltpu.CompilerParams(collective_id=0),
    )(t, W)


# device time: 46689 ns/iter; 1.0022x vs baseline; 1.0022x over previous
import os

import jax
import jax.numpy as jnp
from jax import lax
from jax.experimental import pallas as pl
from jax.experimental.pallas import tpu as pltpu

N_DEV = 4
S = 2
_SKIP_AG = bool(os.environ.get("SKIP_AG"))


def kernel(t, W):
    m_per, k = t.shape
    _, n = W.shape
    chunk = m_per // N_DEV
    sh = chunk // S
    kh = k // 2
    nh = n // 2
    bf16 = jnp.bfloat16
    H = N_DEV - 1

    def body(t_hbm, w_hbm, out_ref,
             t_ref, w_ref, stage, recv, pre_own, pre_comb,
             in_sems, rs_ssem, rs_rsem,
             ag_r_ssem, ag_r_rsem, ag_l_ssem, ag_l_rsem):
        my = lax.axis_index("i")
        left = lax.rem(my + N_DEV - 1, N_DEV)
        right = lax.rem(my + 1, N_DEV)

        cp_t = pltpu.make_async_copy(t_hbm, t_ref, in_sems.at[0])
        cp_t.start()
        cp_w = pltpu.make_async_copy(w_hbm, w_ref, in_sems.at[1])
        cp_w.start()
        cp_t.wait()

        def t_sub(idx, s, lo):
            rows = pl.ds(idx * chunk + s * sh, sh)
            if lo:
                return t_ref[rows, :kh].astype(bf16)
            return t_ref[rows, kh:].astype(bf16)

        def copy(src, dst, ssem, rsem, dev):
            return pltpu.make_async_remote_copy(
                src_ref=src, dst_ref=dst, send_sem=ssem, recv_sem=rsem,
                device_id=(dev,), device_id_type=pl.DeviceIdType.MESH,
            )

        def rs_start(s, j):
            dev = {0: left, 1: right, 2: right, 3: left, 4: left, 5: right}[j]
            c = copy(stage.at[s, j], recv.at[s, j],
                     rs_ssem.at[s, j], rs_rsem.at[s, j], dev)
            c.start()
            return c

        def out_rows(idx, s):
            return pl.ds(idx * chunk + s * sh, sh)

        def start_ag_r(s, h):
            idx = lax.rem(my - h + 2 * N_DEV, N_DEV)
            r = copy(out_ref.at[out_rows(idx, s), pl.ds(0, nh)],
                     out_ref.at[out_rows(idx, s), pl.ds(0, nh)],
                     ag_r_ssem.at[s, h], ag_r_rsem.at[s, h], right)
            r.start()
            return r

        def start_ag_l(s, h):
            idx = lax.rem(my + h, N_DEV)
            l = copy(out_ref.at[out_rows(idx, s), pl.ds(nh, nh)],
                     out_ref.at[out_rows(idx, s), pl.ds(nh, nh)],
                     ag_l_ssem.at[s, h], ag_l_rsem.at[s, h], left)
            l.start()
            return l

        c_m2 = lax.rem(my - 2 + N_DEV, N_DEV)
        c_m1 = lax.rem(my - 1 + N_DEV, N_DEV)
        c_p1 = lax.rem(my + 1, N_DEV)
        for s in range(S):
            stage[s, 0, :, :] = t_sub(c_m2, s, True)
            stage[s, 1, :, :] = t_sub(c_m2, s, False)
            stage[s, 2, :, :] = t_sub(c_p1, s, True)
            stage[s, 3, :, :] = t_sub(c_m1, s, False)

        barrier_sem = pltpu.get_barrier_semaphore()
        for nbr in (left, right):
            pl.semaphore_signal(
                barrier_sem, inc=1,
                device_id=(nbr,), device_id_type=pl.DeviceIdType.MESH,
            )
        pl.semaphore_wait(barrier_sem, 2)

        rs = {}
        for s in range(S):
            rs[s, 0] = rs_start(s, 0)
            rs[s, 1] = rs_start(s, 1)
        for s in range(S):
            rs[s, 2] = rs_start(s, 2)
            rs[s, 3] = rs_start(s, 3)

        for s in range(S):
            pre_comb[s, 0, :, :] = t_sub(c_m1, s, True)
            pre_comb[s, 1, :, :] = t_sub(c_p1, s, False)
            pre_own[s, 0, :, :] = t_sub(my, s, True)
            pre_own[s, 1, :, :] = t_sub(my, s, False)
        cp_w.wait()
        w_bf = w_ref[:, :].astype(bf16)

        for s in range(S):
            rs[s, 0].wait()
            stage[s, 4, :, :] = pre_comb[s, 0] + recv[s, 0]
            rs[s, 4] = rs_start(s, 4)
            rs[s, 1].wait()
            stage[s, 5, :, :] = pre_comb[s, 1] + recv[s, 1]
            rs[s, 5] = rs_start(s, 5)

        ag = {}
        for s in range(S):
            rs[s, 2].wait()
            rs[s, 4].wait()
            sum_lo = pre_own[s, 0] + recv[s, 2] + recv[s, 4]
            rs[s, 3].wait()
            rs[s, 5].wait()
            sum_hi = pre_own[s, 1] + recv[s, 3] + recv[s, 5]
            out_sub = (
                jnp.dot(sum_lo, w_bf[:kh, :],
                        preferred_element_type=jnp.float32)
                + jnp.dot(sum_hi, w_bf[kh:, :],
                          preferred_element_type=jnp.float32)
            ).astype(bf16)
            out_ref[out_rows(my, s), :] = out_sub
            if not _SKIP_AG:
                ag[s, 0] = (start_ag_r(s, 0), start_ag_l(s, 0))

        if not _SKIP_AG:
            for h in range(H - 1):
                for s in range(S):
                    r, l = ag[s, h]
                    r.wait()
                    nr = start_ag_r(s, h + 1)
                    l.wait()
                    nl = start_ag_l(s, h + 1)
                    ag[s, h + 1] = (nr, nl)
            for s in range(S):
                r, l = ag[s, H - 1]
                r.wait()
                l.wait()

    return pl.pallas_call(
        body,
        out_shape=jax.ShapeDtypeStruct((m_per, n), bf16),
        in_specs=[
            pl.BlockSpec(memory_space=pl.ANY),
            pl.BlockSpec(memory_space=pl.ANY),
        ],
        out_specs=pl.BlockSpec(memory_space=pltpu.VMEM),
        scratch_shapes=[
            pltpu.VMEM((m_per, k), t.dtype),
            pltpu.VMEM((k, n), W.dtype),
            pltpu.VMEM((S, 6, sh, kh), bf16),
            pltpu.VMEM((S, 6, sh, kh), bf16),
            pltpu.VMEM((S, 2, sh, kh), bf16),
            pltpu.VMEM((S, 2, sh, kh), bf16),
            pltpu.SemaphoreType.DMA((2,)),
            pltpu.SemaphoreType.DMA((S, 6)),
            pltpu.SemaphoreType.DMA((S, 6)),
            pltpu.SemaphoreType.DMA((S, H)),
            pltpu.SemaphoreType.DMA((S, H)),
            pltpu.SemaphoreType.DMA((S, H)),
            pltpu.SemaphoreType.DMA((S, H)),
        ],
        compiler_params=pltpu.CompilerParams(collective_id=0),
    )(t, W)
